# Initial kernel scaffold; baseline (speedup 1.0000x reference)
import jax
import jax.numpy as jnp
from jax.experimental import pallas as pl


def kernel(x, edge_index, W0, W1, W2, Wfc):
    raise NotImplementedError("write your pallas kernel here")


# speedup vs baseline: 27.5857x; 27.5857x over previous
import functools
import jax
import jax.numpy as jnp
from jax import lax
from jax.experimental import pallas as pl
from jax.experimental.pallas import tpu as pltpu
from jax.experimental.pallas import tpu_sc as plsc

N = 10000
D = 128
NC = 2
NS = 16
NPAD = 10240
RPT = NPAD // NS
CHUNK = 128
NCHUNKS = 80
EPT = NCHUNKS * CHUNK
EPAD = NC * NS * EPT


def _per_tile(sid, fn):
    for k in range(NS):
        @pl.when(sid == k)
        def _():
            fn(k)


def _deg_body(col_hbm, deg_hbm, col_v, ones_v, stage_v, acc_sh):
    cid = lax.axis_index("c")
    sid = lax.axis_index("s")
    base = pl.multiple_of(sid * RPT, RPT)
    pltpu.sync_copy(col_hbm.at[cid, sid], col_v)

    zero16 = jnp.zeros((16,), jnp.float32)
    one16 = jnp.ones((16,), jnp.float32)

    def fills(i, _):
        s = pl.multiple_of(i * 16, 16)
        stage_v[pl.ds(s, 16)] = zero16
        ones_v[pl.ds(s, 16)] = one16
        return 0

    lax.fori_loop(0, RPT // 16, fills, 0, unroll=8)

    _per_tile(sid, lambda k: pltpu.sync_copy(
        stage_v, acc_sh.at[pl.ds(k * RPT, RPT)]))
    plsc.subcore_barrier()

    def hist(j, _):
        pltpu.sync_copy(ones_v.at[pl.ds(0, CHUNK)],
                        acc_sh.at[col_v.at[j]], add=True)
        return 0

    lax.fori_loop(0, NCHUNKS, hist, 0)
    plsc.subcore_barrier()

    _per_tile(sid, lambda k: pltpu.sync_copy(
        acc_sh.at[pl.ds(k * RPT, RPT)], stage_v))
    pltpu.sync_copy(stage_v, deg_hbm.at[cid, pl.ds(base, RPT)])


def _agg_body(xs_hbm, rc_hbm, y_hbm, rc_v, gbuf0, acc_sh, sem0):
    cid = lax.axis_index("c")
    sid = lax.axis_index("s")
    base = pl.multiple_of(sid * RPT, RPT)
    pltpu.sync_copy(rc_hbm.at[cid, sid], rc_v)
    row_v = rc_v.at[0]
    col_v = rc_v.at[1]

    zero16 = jnp.zeros((16,), jnp.float32)

    def zrow(i, _):
        def zcol(g, _):
            gbuf0[i, pl.ds(pl.multiple_of(g * 16, 16), 16)] = zero16
            return 0

        lax.fori_loop(0, D // 16, zcol, 0, unroll=8)
        return 0

    lax.fori_loop(0, CHUNK, zrow, 0)

    def zacc(k):
        for j in range(RPT // CHUNK):
            pltpu.sync_copy(gbuf0, acc_sh.at[pl.ds(k * RPT + j * CHUNK, CHUNK)])

    _per_tile(sid, zacc)
    plsc.subcore_barrier()

    def step(j, _):
        pltpu.async_copy(xs_hbm.at[row_v.at[j]], gbuf0, sem0).wait()
        pltpu.sync_copy(gbuf0, acc_sh.at[col_v.at[j]], add=True)
        return 0

    lax.fori_loop(0, NCHUNKS, step, 0)
    plsc.subcore_barrier()

    def wout(k):
        for j in range(RPT // CHUNK):
            pltpu.sync_copy(acc_sh.at[pl.ds(k * RPT + j * CHUNK, CHUNK)],
                            y_hbm.at[cid, pl.ds(base + j * CHUNK, CHUNK)])

    _per_tile(sid, wout)


def _scale_body(deg_ref, x_ref, xs_ref):
    deg = deg_ref[0, :] + deg_ref[1, :]
    dis = jnp.where(deg > 0, lax.rsqrt(deg), 0.0)
    xs_ref[...] = x_ref[...] * dis[:, None]


def _final_body(x_ref, deg_ref, y_ref, w0_ref, w1_ref, w2_ref, wfc_ref, o_ref,
                m0_ref, m1_ref):
    @pl.when(pl.program_id(0) == 0)
    def _():
        w0 = w0_ref[...]
        w1 = w1_ref[...]
        w2 = w2_ref[...]
        wfc = wfc_ref[...]
        dn = (((0,), (1,)), ((), ()))
        m0_ref[...] = lax.dot_general(w0, wfc[:, 0:D],
                                      dimension_numbers=dn,
                                      preferred_element_type=jnp.float32)
        m1_ref[...] = (
            lax.dot_general(w1, wfc[:, D:2 * D], dimension_numbers=dn,
                            preferred_element_type=jnp.float32)
            + lax.dot_general(w2, wfc[:, 2 * D:3 * D], dimension_numbers=dn,
                              preferred_element_type=jnp.float32))

    deg = deg_ref[0, :] + deg_ref[1, :]
    dis = jnp.where(deg > 0, lax.rsqrt(deg), 0.0)
    y = (y_ref[0] + y_ref[1]) * dis[:, None]
    o_ref[...] = (
        jnp.dot(x_ref[...], m0_ref[...], preferred_element_type=jnp.float32)
        + jnp.dot(y, m1_ref[...], preferred_element_type=jnp.float32))


@functools.lru_cache(maxsize=1)
def _sc_kernels():
    mesh = plsc.VectorSubcoreMesh(
        core_axis_name="c", subcore_axis_name="s",
        num_cores=NC, num_subcores=NS)
    deg_k = pl.kernel(
        _deg_body,
        out_type=jax.ShapeDtypeStruct((NC, NPAD), jnp.float32),
        mesh=mesh,
        scratch_types=[
            pltpu.VMEM((NCHUNKS, CHUNK), jnp.int32),
            pltpu.VMEM((RPT,), jnp.float32),
            pltpu.VMEM((RPT,), jnp.float32),
            pltpu.VMEM_SHARED((NPAD,), jnp.float32),
        ],
    )
    agg_k = pl.kernel(
        _agg_body,
        out_type=jax.ShapeDtypeStruct((NC, NPAD, D), jnp.float32),
        mesh=mesh,
        scratch_types=[
            pltpu.VMEM((2, NCHUNKS, CHUNK), jnp.int32),
            pltpu.VMEM((CHUNK, D), jnp.float32),
            pltpu.VMEM_SHARED((NPAD, D), jnp.float32),
            pltpu.SemaphoreType.DMA,
        ],
    )
    return deg_k, agg_k


def kernel(x, edge_index, W0, W1, W2, Wfc):
    deg_kernel, agg_kernel = _sc_kernels()
    row = edge_index[0]
    col = edge_index[1]
    e = row.shape[0]
    pad = EPAD - e
    pad_idx = (N + jnp.arange(pad, dtype=jnp.int32) % (NPAD - N))
    rowp = jnp.concatenate([row, pad_idx])
    colp = jnp.concatenate([col, pad_idx])
    row4 = rowp.reshape(NC, NS, NCHUNKS, CHUNK)
    col4 = colp.reshape(NC, NS, NCHUNKS, CHUNK)
    rc4 = jnp.stack([row4, col4], axis=2)
    x_pad = jnp.pad(x, ((0, NPAD - N), (0, 0)))

    deg = deg_kernel(col4)

    nb = 10
    rb = NPAD // nb
    xs = pl.pallas_call(
        _scale_body,
        grid=(nb,),
        in_specs=[
            pl.BlockSpec((NC, rb), lambda i: (0, i)),
            pl.BlockSpec((rb, D), lambda i: (i, 0)),
        ],
        out_specs=pl.BlockSpec((rb, D), lambda i: (i, 0)),
        out_shape=jax.ShapeDtypeStruct((NPAD, D), jnp.float32),
    )(deg, x_pad)

    y2 = agg_kernel(xs, rc4)

    out_pad = pl.pallas_call(
        _final_body,
        grid=(nb,),
        in_specs=[
            pl.BlockSpec((rb, D), lambda i: (i, 0)),
            pl.BlockSpec((NC, rb), lambda i: (0, i)),
            pl.BlockSpec((NC, rb, D), lambda i: (0, i, 0)),
            pl.BlockSpec((D, D), lambda i: (0, 0)),
            pl.BlockSpec((D, D), lambda i: (0, 0)),
            pl.BlockSpec((D, D), lambda i: (0, 0)),
            pl.BlockSpec((D, 3 * D), lambda i: (0, 0)),
        ],
        out_specs=pl.BlockSpec((rb, D), lambda i: (i, 0)),
        out_shape=jax.ShapeDtypeStruct((NPAD, D), jnp.float32),
        scratch_shapes=[
            pltpu.VMEM((D, D), jnp.float32),
            pltpu.VMEM((D, D), jnp.float32),
        ],
    )(x_pad, deg, y2, W0, W1, W2, Wfc)

    return out_pad[:N]
